# baseline (speedup 1.0000x reference)
import jax
import jax.numpy as jnp
from jax.experimental import pallas as pl

NSITES = 16384
D = 128
BN = 8192


def _body(x_ref, et_ref, p_ref, o_ref):
    pt = p_ref[:].T
    b0 = x_ref[0:1, :] < 0.0
    b1 = x_ref[1:2, :] < 0.0
    et = et_ref[:]
    e0 = et[:, 0:1]
    e1 = et[:, 1:2]
    e2 = et[:, 2:3]
    e3 = et[:, 3:4]
    sel = jnp.where(b0, jnp.where(b1, e3, e2), jnp.where(b1, e1, e0))
    o_ref[:] = pt + sel


def kernel(x, E, P):
    et = E.T
    return pl.pallas_call(
        _body,
        grid=(NSITES // BN,),
        in_specs=[
            pl.BlockSpec((2, BN), lambda i: (0, i)),
            pl.BlockSpec((D, 4), lambda i: (0, 0)),
            pl.BlockSpec((BN, D), lambda i: (i, 0)),
        ],
        out_specs=pl.BlockSpec((D, BN), lambda i: (0, i)),
        out_shape=jax.ShapeDtypeStruct((D, NSITES), jnp.float32),
    )(x, et, P)


# speedup vs baseline: 1.0504x; 1.0504x over previous
import jax
import jax.numpy as jnp
from jax.experimental import pallas as pl

NSITES = 16384
D = 128
BN = 4096


def _body(x_ref, et_ref, p_ref, o_ref):
    pt = p_ref[:].T
    b0 = x_ref[0:1, :] < 0.0
    b1 = x_ref[1:2, :] < 0.0
    et = et_ref[:]
    e0 = et[:, 0:1]
    e1 = et[:, 1:2]
    e2 = et[:, 2:3]
    e3 = et[:, 3:4]
    sel = jnp.where(b0, jnp.where(b1, e3, e2), jnp.where(b1, e1, e0))
    o_ref[:] = pt + sel


def kernel(x, E, P):
    et = E.T
    return pl.pallas_call(
        _body,
        grid=(NSITES // BN,),
        in_specs=[
            pl.BlockSpec((2, BN), lambda i: (0, i)),
            pl.BlockSpec((D, 4), lambda i: (0, 0)),
            pl.BlockSpec((BN, D), lambda i: (i, 0)),
        ],
        out_specs=pl.BlockSpec((D, BN), lambda i: (0, i)),
        out_shape=jax.ShapeDtypeStruct((D, NSITES), jnp.float32),
    )(x, et, P)
